# Initial kernel scaffold; baseline (speedup 1.0000x reference)
import jax
import jax.numpy as jnp
from jax.experimental import pallas as pl


def kernel(x, edge_index, Wa, ba, W1, b1, W2, b2):
    raise NotImplementedError("write your pallas kernel here")


# speedup vs baseline: 8.6892x; 8.6892x over previous
import functools

import jax
import jax.numpy as jnp
from jax import lax
from jax.experimental import pallas as pl
from jax.experimental.pallas import tpu as pltpu
from jax.experimental.pallas import tpu_sc as plsc

NC = 2
NS = 16
NW = NC * NS
CHUNK = 128


def _pad_n(n):
    q = NS * CHUNK
    return -(-n // q) * q


def _make_prop(n, e, d):
    nchunks = e // CHUNK
    n_pad = _pad_n(n)
    rows_per = n_pad // NS
    nfull = rows_per // CHUNK
    mesh = plsc.VectorSubcoreMesh(core_axis_name="c", subcore_axis_name="s")

    @functools.partial(
        pl.kernel,
        mesh=mesh,
        out_type=jax.ShapeDtypeStruct((NC, n_pad, d), jnp.float32),
        scratch_types=[
            pltpu.VMEM((CHUNK,), jnp.int32),
            pltpu.VMEM((CHUNK,), jnp.int32),
            pltpu.VMEM((CHUNK, d), jnp.float32),
            pltpu.VMEM_SHARED((n_pad, d), jnp.float32),
            pltpu.SemaphoreType.DMA,
        ],
    )
    def prop(g_hbm, src_hbm, dst_hbm, out_hbm, src_v, dst_v, rows_v, acc_sh,
             sem):
        cid = lax.axis_index("c")
        sid = lax.axis_index("s")
        wid = sid * NC + cid

        def zrow(i, carry):
            def zlane(j, c2):
                rows_v[i, pl.ds(j * 16, 16)] = jnp.zeros((16,), jnp.float32)
                return c2
            return lax.fori_loop(0, d // 16, zlane, carry)
        lax.fori_loop(0, CHUNK, zrow, 0)

        base_r = pl.multiple_of(sid * rows_per, CHUNK)

        def zcopy(i, carry):
            pltpu.sync_copy(rows_v,
                            acc_sh.at[pl.ds(base_r + i * CHUNK, CHUNK), :])
            return carry
        lax.fori_loop(0, nfull, zcopy, 0)
        plsc.subcore_barrier()

        c0 = wid * nchunks // NW
        c1 = (wid + 1) * nchunks // NW

        def body(c, carry):
            ebase = pl.multiple_of(c * CHUNK, CHUNK)
            pltpu.sync_copy(src_hbm.at[pl.ds(ebase, CHUNK)], src_v)
            pltpu.sync_copy(dst_hbm.at[pl.ds(ebase, CHUNK)], dst_v)
            pltpu.async_copy(g_hbm.at[src_v], rows_v, sem).wait()
            pltpu.sync_copy(rows_v, acc_sh.at[dst_v], add=True)
            return carry
        lax.fori_loop(c0, c1, body, 0)
        plsc.subcore_barrier()

        def wcopy(i, carry):
            pltpu.sync_copy(acc_sh.at[pl.ds(base_r + i * CHUNK, CHUNK), :],
                            out_hbm.at[cid,
                                       pl.ds(base_r + i * CHUNK, CHUNK), :])
            return carry
        lax.fori_loop(0, nfull, wcopy, 0)

    return prop


def _make_deg(n, e):
    nchunks = e // CHUNK
    n_pad = _pad_n(n)
    zch = n_pad // NS
    mesh = plsc.VectorSubcoreMesh(core_axis_name="c", subcore_axis_name="s")

    @functools.partial(
        pl.kernel,
        mesh=mesh,
        out_type=jax.ShapeDtypeStruct((NC, n_pad), jnp.float32),
        scratch_types=[
            pltpu.VMEM((CHUNK,), jnp.int32),
            pltpu.VMEM((CHUNK,), jnp.float32),
            pltpu.VMEM((zch,), jnp.float32),
            pltpu.VMEM_SHARED((n_pad,), jnp.float32),
        ],
    )
    def degk(dst_hbm, out_hbm, dst_v, ones_v, zbuf, deg_sh):
        cid = lax.axis_index("c")
        sid = lax.axis_index("s")
        wid = sid * NC + cid

        def fill(i, carry):
            zbuf[pl.ds(i * 16, 16)] = jnp.zeros((16,), jnp.float32)
            return carry
        lax.fori_loop(0, zch // 16, fill, 0)

        def fones(i, carry):
            ones_v[pl.ds(i * 16, 16)] = jnp.ones((16,), jnp.float32)
            return carry
        lax.fori_loop(0, CHUNK // 16, fones, 0)

        base_r = pl.multiple_of(sid * zch, CHUNK)
        pltpu.sync_copy(zbuf, deg_sh.at[pl.ds(base_r, zch)])
        plsc.subcore_barrier()

        c0 = wid * nchunks // NW
        c1 = (wid + 1) * nchunks // NW

        def body(c, carry):
            ebase = pl.multiple_of(c * CHUNK, CHUNK)
            pltpu.sync_copy(dst_hbm.at[pl.ds(ebase, CHUNK)], dst_v)
            pltpu.sync_copy(ones_v, deg_sh.at[dst_v], add=True)
            return carry
        lax.fori_loop(c0, c1, body, 0)
        plsc.subcore_barrier()

        pltpu.sync_copy(deg_sh.at[pl.ds(base_r, zch)],
                        out_hbm.at[cid, pl.ds(base_r, zch)])

    return degk


ROWS = 256


def _dis(degp_ref):
    r0 = pl.program_id(0) * ROWS
    deg = degp_ref[0, pl.ds(r0, ROWS)] + degp_ref[1, pl.ds(r0, ROWS)]
    return jnp.where(deg > 0, lax.rsqrt(deg), 0.0)


def _pre_body(x_ref, wa_ref, ba_ref, degp_ref, h0_ref, g0_ref):
    x = x_ref[...]
    dis = _dis(degp_ref)
    logits = jnp.dot(x, wa_ref[...], preferred_element_type=jnp.float32)
    logits = logits + ba_ref[...]
    m = jnp.max(logits, axis=1, keepdims=True)
    ex = jnp.exp(logits - m)
    sm = ex / jnp.sum(ex, axis=1, keepdims=True)
    h0 = x * sm
    h0_ref[...] = h0
    g0_ref[...] = h0 * dis[:, None]


def _scale_body(ap_ref, degp_ref, g1_ref):
    dis = _dis(degp_ref)
    a = ap_ref[0] + ap_ref[1]
    g1_ref[...] = a * (dis * dis)[:, None]


def _mm1_body(h0_ref, a0p_ref, a1p_ref, degp_ref, w_ref, b_ref,
              out1_ref, g0b_ref):
    dis = _dis(degp_ref)
    h1 = (a0p_ref[0] + a0p_ref[1]) * dis[:, None]
    h2 = (a1p_ref[0] + a1p_ref[1]) * dis[:, None]
    z = (jnp.dot(h0_ref[...], w_ref[0], preferred_element_type=jnp.float32)
         + jnp.dot(h1, w_ref[1], preferred_element_type=jnp.float32)
         + jnp.dot(h2, w_ref[2], preferred_element_type=jnp.float32)
         + b_ref[...])
    o = jnp.maximum(z, 0.0)
    out1_ref[...] = o
    g0b_ref[...] = o * dis[:, None]


def _mm2_body(h0_ref, a0p_ref, a1p_ref, degp_ref, w_ref, b_ref, out_ref):
    dis = _dis(degp_ref)
    h1 = (a0p_ref[0] + a0p_ref[1]) * dis[:, None]
    h2 = (a1p_ref[0] + a1p_ref[1]) * dis[:, None]
    z = (jnp.dot(h0_ref[...], w_ref[0], preferred_element_type=jnp.float32)
         + jnp.dot(h1, w_ref[1], preferred_element_type=jnp.float32)
         + jnp.dot(h2, w_ref[2], preferred_element_type=jnp.float32)
         + b_ref[...])
    m = jnp.max(z, axis=1, keepdims=True)
    lse = m + jnp.log(jnp.sum(jnp.exp(z - m), axis=1, keepdims=True))
    out_ref[...] = z - lse


def _row_spec(d):
    return pl.BlockSpec((ROWS, d), lambda i: (i, 0))


def _part_spec(d):
    return pl.BlockSpec((NC, ROWS, d), lambda i: (0, i, 0))


def _deg_spec(n_pad):
    return pl.BlockSpec((NC, n_pad), lambda i: (0, 0))


def _full(shape):
    nd = len(shape)
    return pl.BlockSpec(shape, lambda i, _n=nd: (0,) * _n)


def kernel(x, edge_index, Wa, ba, W1, b1, W2, b2):
    n, d_in = x.shape
    e = edge_index.shape[1]
    hid = W1.shape[2]
    d_out = W2.shape[2]
    n_pad = _pad_n(n)
    src = edge_index[0]
    dst = edge_index[1]
    xp = jnp.pad(x, ((0, n_pad - n), (0, 0)))
    grid = (n_pad // ROWS,)

    degp = _make_deg(n, e)(dst)

    prop = _make_prop(n, e, d_in)

    h0, g0 = pl.pallas_call(
        _pre_body,
        grid=grid,
        in_specs=[_row_spec(d_in), _full(Wa.shape), _full((1, d_in)),
                  _deg_spec(n_pad)],
        out_specs=[_row_spec(d_in), _row_spec(d_in)],
        out_shape=[jax.ShapeDtypeStruct((n_pad, d_in), jnp.float32)] * 2,
    )(xp, Wa, ba.reshape(1, -1), degp)

    a0p = prop(g0, src, dst)
    g1 = pl.pallas_call(
        _scale_body,
        grid=grid,
        in_specs=[_part_spec(d_in), _deg_spec(n_pad)],
        out_specs=_row_spec(d_in),
        out_shape=jax.ShapeDtypeStruct((n_pad, d_in), jnp.float32),
    )(a0p, degp)
    a1p = prop(g1, src, dst)

    out1, g0b = pl.pallas_call(
        _mm1_body,
        grid=grid,
        in_specs=[_row_spec(d_in), _part_spec(d_in), _part_spec(d_in),
                  _deg_spec(n_pad), _full(W1.shape), _full((1, hid))],
        out_specs=[_row_spec(hid), _row_spec(hid)],
        out_shape=[jax.ShapeDtypeStruct((n_pad, hid), jnp.float32)] * 2,
    )(h0, a0p, a1p, degp, W1, b1.reshape(1, -1))

    b0p = prop(g0b, src, dst)
    g1b = pl.pallas_call(
        _scale_body,
        grid=grid,
        in_specs=[_part_spec(hid), _deg_spec(n_pad)],
        out_specs=_row_spec(hid),
        out_shape=jax.ShapeDtypeStruct((n_pad, hid), jnp.float32),
    )(b0p, degp)
    b1p = prop(g1b, src, dst)

    out = pl.pallas_call(
        _mm2_body,
        grid=grid,
        in_specs=[_row_spec(hid), _part_spec(hid), _part_spec(hid),
                  _deg_spec(n_pad), _full(W2.shape), _full((1, d_out))],
        out_specs=_row_spec(d_out),
        out_shape=jax.ShapeDtypeStruct((n_pad, d_out), jnp.float32),
    )(out1, b0p, b1p, degp, W2, b2.reshape(1, -1))
    return out[:n]
